# initial kernel scaffold (unmeasured)
import jax
import jax.numpy as jnp
from jax import lax
from jax.experimental import pallas as pl
from jax.experimental.pallas import tpu as pltpu

N_Z = 4
T = 4096
D = 2048
V_SHARD = 8192
C = T // N_Z


def _ring_allreduce_z(partial):

    def body(p_hbm, out_hbm, send_buf, recv_buf, send_sems, recv_sems,
             credit_sem, load_sem, store_sem):
        my_x = lax.axis_index("x")
        my_y = lax.axis_index("y")
        my_z = lax.axis_index("z")
        right = lax.rem(my_z + 1, N_Z)
        left = lax.rem(my_z + N_Z - 1, N_Z)

        barrier = pltpu.get_barrier_semaphore()
        for nbr in (left, right):
            pl.semaphore_signal(
                barrier, inc=1, device_id=(my_x, my_y, nbr),
                device_id_type=pl.DeviceIdType.MESH,
            )
        pl.semaphore_wait(barrier, 2)

        def rows(c):
            return pl.ds(c * C, C)

        def credit_left():
            pl.semaphore_signal(
                credit_sem, inc=1, device_id=(my_x, my_y, left),
                device_id_type=pl.DeviceIdType.MESH,
            )

        def store_out(src_ref, c):
            cp = pltpu.make_async_copy(src_ref, out_hbm.at[rows(c), :], store_sem)
            cp.start()
            cp.wait()

        def send_step(slot, src_ref):
            rdma = pltpu.make_async_remote_copy(
                src_ref=src_ref,
                dst_ref=recv_buf.at[slot],
                send_sem=send_sems.at[slot],
                recv_sem=recv_sems.at[slot],
                device_id=(my_x, my_y, right),
                device_id_type=pl.DeviceIdType.MESH,
            )
            rdma.start()
            return rdma

        for s in range(N_Z):
            slot = s % 2
            chunk = lax.rem(my_z + 2 * N_Z - s, N_Z)
            cp = pltpu.make_async_copy(
                p_hbm.at[rows(chunk), :], send_buf.at[slot], load_sem)
            cp.start()
            cp.wait()
            if s > 0:
                send_buf[slot, :, :] = (
                    send_buf[slot, :, :] + recv_buf[(s - 1) % 2, :, :])
                credit_left()
            if s >= 2:
                pl.semaphore_wait(credit_sem, 1)
            rdma = send_step(slot, send_buf.at[slot])
            if s == N_Z - 1:
                store_out(send_buf.at[slot], chunk)
            rdma.wait()

        for s in range(N_Z, 2 * N_Z - 2):
            slot = s % 2
            prev = (s - 1) % 2
            c_fwd = lax.rem(my_z + 2 * N_Z - (s - 3), N_Z)
            pl.semaphore_wait(credit_sem, 1)
            rdma = send_step(slot, recv_buf.at[prev])
            store_out(recv_buf.at[prev], c_fwd)
            rdma.wait()
            if s < 2 * N_Z - 3:
                credit_left()

        final_slot = (2 * N_Z - 3) % 2
        store_out(recv_buf.at[final_slot], lax.rem(my_z + 2, N_Z))

    return pl.pallas_call(
        body,
        out_shape=jax.ShapeDtypeStruct((T, D), jnp.float32),
        in_specs=[pl.BlockSpec(memory_space=pltpu.ANY)],
        out_specs=pl.BlockSpec(memory_space=pltpu.ANY),
        scratch_shapes=[
            pltpu.VMEM((2, C, D), jnp.float32),
            pltpu.VMEM((2, C, D), jnp.float32),
            pltpu.SemaphoreType.DMA((2,)),
            pltpu.SemaphoreType.DMA((2,)),
            pltpu.SemaphoreType.REGULAR,
            pltpu.SemaphoreType.DMA,
            pltpu.SemaphoreType.DMA,
        ],
        compiler_params=pltpu.CompilerParams(collective_id=0),
    )(partial)


def kernel(ids, E):
    my_z = lax.axis_index("z")
    local = ids - my_z * V_SHARD
    valid = (local >= 0) & (local < V_SHARD)
    safe = jnp.clip(local, 0, V_SHARD - 1)
    partial = jnp.take(E, safe, axis=0) * valid[:, None].astype(jnp.float32)
    return _ring_allreduce_z(partial)


# baseline (device time: 3383203 ns/iter reference)
import jax
import jax.numpy as jnp
from jax import lax
from jax.experimental import pallas as pl
from jax.experimental.pallas import tpu as pltpu

N_Z = 4
T = 4096
D = 2048
V_SHARD = 8192
C = T // N_Z


def _ring_allreduce_z(partial):

    def body(p_hbm, out_hbm, send_buf, recv_buf, send_sems, recv_sems,
             credit_sem, load_sem, store_sem):
        my_x = lax.axis_index("x")
        my_y = lax.axis_index("y")
        my_z = lax.axis_index("z")
        right = lax.rem(my_z + 1, N_Z)
        left = lax.rem(my_z + N_Z - 1, N_Z)

        barrier = pltpu.get_barrier_semaphore()
        for nbr in (left, right):
            pl.semaphore_signal(
                barrier, inc=1, device_id=(my_x, my_y, nbr),
                device_id_type=pl.DeviceIdType.MESH,
            )
        pl.semaphore_wait(barrier, 2)

        def rows(c):
            return pl.ds(c * C, C)

        def credit_left():
            pl.semaphore_signal(
                credit_sem, inc=1, device_id=(my_x, my_y, left),
                device_id_type=pl.DeviceIdType.MESH,
            )

        def store_out(src_ref, c):
            cp = pltpu.make_async_copy(src_ref, out_hbm.at[rows(c), :], store_sem)
            cp.start()
            cp.wait()

        def send_step(slot, src_ref):
            rdma = pltpu.make_async_remote_copy(
                src_ref=src_ref,
                dst_ref=recv_buf.at[slot],
                send_sem=send_sems.at[slot],
                recv_sem=recv_sems.at[slot],
                device_id=(my_x, my_y, right),
                device_id_type=pl.DeviceIdType.MESH,
            )
            rdma.start()
            return rdma

        for s in range(N_Z):
            slot = s % 2
            chunk = lax.rem(my_z + 2 * N_Z - s, N_Z)
            cp = pltpu.make_async_copy(
                p_hbm.at[rows(chunk), :], send_buf.at[slot], load_sem)
            cp.start()
            cp.wait()
            if s > 0:
                send_buf[slot, :, :] = (
                    send_buf[slot, :, :] + recv_buf[(s - 1) % 2, :, :])
                credit_left()
            if s >= 2:
                pl.semaphore_wait(credit_sem, 1)
            rdma = send_step(slot, send_buf.at[slot])
            if s == N_Z - 1:
                store_out(send_buf.at[slot], chunk)
            rdma.wait()

        for s in range(N_Z, 2 * N_Z - 2):
            slot = s % 2
            prev = (s - 1) % 2
            c_fwd = lax.rem(my_z + 2 * N_Z - (s - 4), N_Z)
            pl.semaphore_wait(credit_sem, 1)
            rdma = send_step(slot, recv_buf.at[prev])
            store_out(recv_buf.at[prev], c_fwd)
            rdma.wait()
            if s < 2 * N_Z - 3:
                credit_left()

        final_slot = (2 * N_Z - 3) % 2
        store_out(recv_buf.at[final_slot], lax.rem(my_z + 2, N_Z))

    return pl.pallas_call(
        body,
        out_shape=jax.ShapeDtypeStruct((T, D), jnp.float32),
        in_specs=[pl.BlockSpec(memory_space=pl.ANY)],
        out_specs=pl.BlockSpec(memory_space=pl.ANY),
        scratch_shapes=[
            pltpu.VMEM((2, C, D), jnp.float32),
            pltpu.VMEM((2, C, D), jnp.float32),
            pltpu.SemaphoreType.DMA((2,)),
            pltpu.SemaphoreType.DMA((2,)),
            pltpu.SemaphoreType.REGULAR,
            pltpu.SemaphoreType.DMA,
            pltpu.SemaphoreType.DMA,
        ],
        compiler_params=pltpu.CompilerParams(collective_id=0),
    )(partial)


def kernel(ids, E):
    my_z = lax.axis_index("z")
    local = ids - my_z * V_SHARD
    valid = (local >= 0) & (local < V_SHARD)
    safe = jnp.clip(local, 0, V_SHARD - 1)
    partial = jnp.take(E, safe, axis=0) * valid[:, None].astype(jnp.float32)
    return _ring_allreduce_z(partial)


# device time: 772509 ns/iter; 4.3795x vs baseline; 4.3795x over previous
import jax
import jax.numpy as jnp
from jax import lax
from jax.experimental import pallas as pl
from jax.experimental.pallas import tpu as pltpu

N_Z = 4
T = 4096
D = 2048
V_SHARD = 8192
C = T // N_Z


def _ring_allreduce_z(partial):

    def body(p_hbm, out_hbm, send_buf, recv_buf, send_sems, recv_sems,
             credit_sem, load_sem, store_sem):
        my_x = lax.axis_index("x")
        my_y = lax.axis_index("y")
        my_z = lax.axis_index("z")
        right = lax.rem(my_z + 1, N_Z)
        left = lax.rem(my_z + N_Z - 1, N_Z)

        barrier = pltpu.get_barrier_semaphore()
        for nbr in (left, right):
            pl.semaphore_signal(
                barrier, inc=1, device_id=(my_x, my_y, nbr),
                device_id_type=pl.DeviceIdType.MESH,
            )
        pl.semaphore_wait(barrier, 2)

        def rows(c):
            return pl.ds(c * C, C)

        def credit_left():
            pl.semaphore_signal(
                credit_sem, inc=1, device_id=(my_x, my_y, left),
                device_id_type=pl.DeviceIdType.MESH,
            )

        def store_out(src_ref, c):
            cp = pltpu.make_async_copy(src_ref, out_hbm.at[rows(c), :], store_sem)
            cp.start()
            cp.wait()

        def send_step(slot, src_ref):
            rdma = pltpu.make_async_remote_copy(
                src_ref=src_ref,
                dst_ref=recv_buf.at[slot],
                send_sem=send_sems.at[slot],
                recv_sem=recv_sems.at[slot],
                device_id=(my_x, my_y, right),
                device_id_type=pl.DeviceIdType.MESH,
            )
            rdma.start()
            return rdma

        for s in range(N_Z):
            slot = s % 2
            chunk = lax.rem(my_z + 2 * N_Z - s, N_Z)
            cp = pltpu.make_async_copy(
                p_hbm.at[rows(chunk), :], send_buf.at[slot], load_sem)
            cp.start()
            cp.wait()
            if s > 0:
                send_buf[slot, :, :] = (
                    send_buf[slot, :, :] + recv_buf[(s - 1) % 2, :, :])
                credit_left()
            if s >= 2:
                pl.semaphore_wait(credit_sem, 1)
            rdma = send_step(slot, send_buf.at[slot])
            if s == N_Z - 1:
                store_out(send_buf.at[slot], chunk)
            rdma.wait()

        for s in range(N_Z, 2 * N_Z - 2):
            slot = s % 2
            prev = (s - 1) % 2
            c_fwd = lax.rem(my_z + 2 * N_Z - (s - 4), N_Z)
            pl.semaphore_wait(credit_sem, 1)
            rdma = send_step(slot, recv_buf.at[prev])
            store_out(recv_buf.at[prev], c_fwd)
            rdma.wait()
            if s < 2 * N_Z - 3:
                credit_left()

        final_slot = (2 * N_Z - 3) % 2
        store_out(recv_buf.at[final_slot], lax.rem(my_z + 2, N_Z))

    return pl.pallas_call(
        body,
        out_shape=jax.ShapeDtypeStruct((T, D), jnp.float32),
        in_specs=[pl.BlockSpec(memory_space=pl.ANY)],
        out_specs=pl.BlockSpec(memory_space=pl.ANY),
        scratch_shapes=[
            pltpu.VMEM((2, C, D), jnp.float32),
            pltpu.VMEM((2, C, D), jnp.float32),
            pltpu.SemaphoreType.DMA((2,)),
            pltpu.SemaphoreType.DMA((2,)),
            pltpu.SemaphoreType.REGULAR,
            pltpu.SemaphoreType.DMA,
            pltpu.SemaphoreType.DMA,
        ],
        compiler_params=pltpu.CompilerParams(collective_id=0),
    )(partial)


def kernel(ids, E):
    my_z = lax.axis_index("z")
    local = ids - my_z * V_SHARD
    one_hot = (local[:, None] == lax.iota(jnp.int32, V_SHARD)[None, :])
    partial = jnp.dot(
        one_hot.astype(jnp.bfloat16), E.astype(jnp.bfloat16),
        preferred_element_type=jnp.float32)
    return _ring_allreduce_z(partial)


# device time: 453588 ns/iter; 7.4588x vs baseline; 1.7031x over previous
import jax
import jax.numpy as jnp
from jax import lax
from jax.experimental import pallas as pl
from jax.experimental.pallas import tpu as pltpu

N_Z = 4
T = 4096
D = 2048
V_SHARD = 8192
C = T // N_Z
W = D // 2


def _allreduce(partial):

    def body(p_hbm, out_hbm, send_buf, recv_buf, x_buf,
             send_sems, recv_sems, x_send_sems, x_recv_sems,
             credit_sem, load_sem, store_sem):
        my_x = lax.axis_index("x")
        my_y = lax.axis_index("y")
        my_z = lax.axis_index("z")
        right = lax.rem(my_z + 1, N_Z)
        left = lax.rem(my_z + N_Z - 1, N_Z)
        partner = (1 - my_x, my_y, my_z)

        barrier = pltpu.get_barrier_semaphore()
        for dev in ((my_x, my_y, left), (my_x, my_y, right), partner):
            pl.semaphore_signal(
                barrier, inc=1, device_id=dev,
                device_id_type=pl.DeviceIdType.MESH,
            )
        pl.semaphore_wait(barrier, 3)

        def rows(c):
            return pl.ds(c * C, C)

        my_cols = pl.ds(my_x * W, W)
        partner_cols = pl.ds((1 - my_x) * W, W)

        def credit_left():
            pl.semaphore_signal(
                credit_sem, inc=1, device_id=(my_x, my_y, left),
                device_id_type=pl.DeviceIdType.MESH,
            )

        def store_out(src_ref, c):
            cp = pltpu.make_async_copy(
                src_ref, out_hbm.at[rows(c), my_cols], store_sem)
            cp.start()
            cp.wait()

        def send_step(slot, src_ref):
            rdma = pltpu.make_async_remote_copy(
                src_ref=src_ref,
                dst_ref=recv_buf.at[slot],
                send_sem=send_sems.at[slot],
                recv_sem=recv_sems.at[slot],
                device_id=(my_x, my_y, right),
                device_id_type=pl.DeviceIdType.MESH,
            )
            rdma.start()
            return rdma

        x_rdmas = []

        def x_send(j, src_ref, c):
            rdma = pltpu.make_async_remote_copy(
                src_ref=src_ref,
                dst_ref=out_hbm.at[rows(c), my_cols],
                send_sem=x_send_sems.at[j],
                recv_sem=x_recv_sems.at[j],
                device_id=partner,
                device_id_type=pl.DeviceIdType.MESH,
            )
            rdma.start()
            x_rdmas.append(rdma)

        for s in range(N_Z):
            slot = s % 2
            chunk = lax.rem(my_z + 2 * N_Z - s, N_Z)
            cp = pltpu.make_async_copy(
                p_hbm.at[rows(chunk), :], send_buf.at[slot], load_sem)
            cp.start()
            cp.wait()
            if s > 0:
                send_buf[slot, :, :] = (
                    send_buf[slot, :, :] + recv_buf[(s - 1) % 2, :, :])
                credit_left()
            if s >= 2:
                pl.semaphore_wait(credit_sem, 1)
            rdma = send_step(slot, send_buf.at[slot])
            if s == N_Z - 1:
                x_send(0, send_buf.at[slot], chunk)
                store_out(send_buf.at[slot], chunk)
            rdma.wait()

        for s in range(N_Z, 2 * N_Z - 2):
            slot = s % 2
            prev = (s - 1) % 2
            c_fwd = lax.rem(my_z + 2 * N_Z - (s - 4), N_Z)
            pl.semaphore_wait(credit_sem, 1)
            rdma = send_step(slot, recv_buf.at[prev])
            if s == N_Z:
                x_buf[:, :] = recv_buf[prev, :, :]
                x_send(1, x_buf, c_fwd)
            else:
                x_send(2, recv_buf.at[prev], c_fwd)
            store_out(recv_buf.at[prev], c_fwd)
            rdma.wait()
            if s < 2 * N_Z - 3:
                credit_left()

        final_slot = (2 * N_Z - 3) % 2
        c_last = lax.rem(my_z + 2, N_Z)
        x_send(3, recv_buf.at[final_slot], c_last)
        store_out(recv_buf.at[final_slot], c_last)

        for rdma in x_rdmas:
            rdma.wait_send()
        c_sched = [lax.rem(my_z + k, N_Z) for k in (1, 0, 3, 2)]
        for j, c in enumerate(c_sched):
            recv = pltpu.make_async_remote_copy(
                src_ref=x_buf,
                dst_ref=out_hbm.at[rows(c), partner_cols],
                send_sem=x_send_sems.at[j],
                recv_sem=x_recv_sems.at[j],
                device_id=partner,
                device_id_type=pl.DeviceIdType.MESH,
            )
            recv.wait_recv()

    return pl.pallas_call(
        body,
        out_shape=jax.ShapeDtypeStruct((T, D), jnp.float32),
        in_specs=[pl.BlockSpec(memory_space=pl.ANY)],
        out_specs=pl.BlockSpec(memory_space=pl.ANY),
        scratch_shapes=[
            pltpu.VMEM((2, C, W), jnp.float32),
            pltpu.VMEM((2, C, W), jnp.float32),
            pltpu.VMEM((C, W), jnp.float32),
            pltpu.SemaphoreType.DMA((2,)),
            pltpu.SemaphoreType.DMA((2,)),
            pltpu.SemaphoreType.DMA((4,)),
            pltpu.SemaphoreType.DMA((4,)),
            pltpu.SemaphoreType.REGULAR,
            pltpu.SemaphoreType.DMA,
            pltpu.SemaphoreType.DMA,
        ],
        compiler_params=pltpu.CompilerParams(collective_id=0),
    )(partial)


def kernel(ids, E):
    my_x = lax.axis_index("x")
    my_z = lax.axis_index("z")
    local = ids - my_z * V_SHARD
    e_half = lax.dynamic_slice_in_dim(E, my_x * W, W, axis=1)
    one_hot = (local[:, None] == lax.iota(jnp.int32, V_SHARD)[None, :])
    partial = jnp.dot(
        one_hot.astype(jnp.bfloat16), e_half.astype(jnp.bfloat16),
        preferred_element_type=jnp.float32)
    return _allreduce(partial)


# device time: 328646 ns/iter; 10.2944x vs baseline; 1.3802x over previous
import jax
import jax.numpy as jnp
from jax import lax
from jax.experimental import pallas as pl
from jax.experimental.pallas import tpu as pltpu

N_Z = 4
T = 4096
D = 2048
V_SHARD = 8192
C = T // N_Z
W = D // 4


def _allreduce(partial):

    def body(p_hbm, out_hbm, send_buf, recv_buf, x_buf,
             send_sems, recv_sems, x_send_sems, x_recv_sems,
             credit_sem, load_sem, store_sem):
        my_x = lax.axis_index("x")
        my_y = lax.axis_index("y")
        my_z = lax.axis_index("z")
        right = lax.rem(my_z + 1, N_Z)
        left = lax.rem(my_z + N_Z - 1, N_Z)
        ybit = lax.rem(my_y, 2)
        yp = my_y + 1 - 2 * ybit
        my_q = my_x * 2 + ybit
        partners = (
            ((1 - my_x, my_y, my_z), (1 - my_x) * 2 + ybit),
            ((my_x, yp, my_z), my_x * 2 + (1 - ybit)),
            ((1 - my_x, yp, my_z), (1 - my_x) * 2 + (1 - ybit)),
        )

        barrier = pltpu.get_barrier_semaphore()
        for dev in ((my_x, my_y, left), (my_x, my_y, right),
                    *(p[0] for p in partners)):
            pl.semaphore_signal(
                barrier, inc=1, device_id=dev,
                device_id_type=pl.DeviceIdType.MESH,
            )
        pl.semaphore_wait(barrier, 5)

        def rows(c):
            return pl.ds(c * C, C)

        my_cols = pl.ds(my_q * W, W)

        def credit_left():
            pl.semaphore_signal(
                credit_sem, inc=1, device_id=(my_x, my_y, left),
                device_id_type=pl.DeviceIdType.MESH,
            )

        def store_out(src_ref, c):
            cp = pltpu.make_async_copy(
                src_ref, out_hbm.at[rows(c), my_cols], store_sem)
            cp.start()
            cp.wait()

        def send_step(slot, src_ref):
            rdma = pltpu.make_async_remote_copy(
                src_ref=src_ref,
                dst_ref=recv_buf.at[slot],
                send_sem=send_sems.at[slot],
                recv_sem=recv_sems.at[slot],
                device_id=(my_x, my_y, right),
                device_id_type=pl.DeviceIdType.MESH,
            )
            rdma.start()
            return rdma

        x_rdmas = []

        def x_send(j, src_ref, c):
            for t, (dev, _q) in enumerate(partners):
                rdma = pltpu.make_async_remote_copy(
                    src_ref=src_ref,
                    dst_ref=out_hbm.at[rows(c), my_cols],
                    send_sem=x_send_sems.at[j, t],
                    recv_sem=x_recv_sems.at[j, t],
                    device_id=dev,
                    device_id_type=pl.DeviceIdType.MESH,
                )
                rdma.start()
                x_rdmas.append(rdma)

        for s in range(N_Z):
            slot = s % 2
            chunk = lax.rem(my_z + 2 * N_Z - s, N_Z)
            cp = pltpu.make_async_copy(
                p_hbm.at[rows(chunk), :], send_buf.at[slot], load_sem)
            cp.start()
            cp.wait()
            if s > 0:
                send_buf[slot, :, :] = (
                    send_buf[slot, :, :] + recv_buf[(s - 1) % 2, :, :])
                credit_left()
            if s >= 2:
                pl.semaphore_wait(credit_sem, 1)
            rdma = send_step(slot, send_buf.at[slot])
            if s == N_Z - 1:
                x_send(0, send_buf.at[slot], chunk)
                store_out(send_buf.at[slot], chunk)
            rdma.wait()

        for s in range(N_Z, 2 * N_Z - 2):
            slot = s % 2
            prev = (s - 1) % 2
            c_fwd = lax.rem(my_z + 2 * N_Z - (s - 4), N_Z)
            pl.semaphore_wait(credit_sem, 1)
            rdma = send_step(slot, recv_buf.at[prev])
            if s == N_Z:
                x_buf[:, :] = recv_buf[prev, :, :]
                x_send(1, x_buf, c_fwd)
            else:
                x_send(2, recv_buf.at[prev], c_fwd)
            store_out(recv_buf.at[prev], c_fwd)
            rdma.wait()
            if s < 2 * N_Z - 3:
                credit_left()

        final_slot = (2 * N_Z - 3) % 2
        c_last = lax.rem(my_z + 2, N_Z)
        x_send(3, recv_buf.at[final_slot], c_last)
        store_out(recv_buf.at[final_slot], c_last)

        for rdma in x_rdmas:
            rdma.wait_send()
        c_sched = [lax.rem(my_z + k, N_Z) for k in (1, 0, 3, 2)]
        for t, (dev, q) in enumerate(partners):
            for j, c in enumerate(c_sched):
                recv = pltpu.make_async_remote_copy(
                    src_ref=x_buf,
                    dst_ref=out_hbm.at[rows(c), pl.ds(q * W, W)],
                    send_sem=x_send_sems.at[j, t],
                    recv_sem=x_recv_sems.at[j, t],
                    device_id=dev,
                    device_id_type=pl.DeviceIdType.MESH,
                )
                recv.wait_recv()

    return pl.pallas_call(
        body,
        out_shape=jax.ShapeDtypeStruct((T, D), jnp.float32),
        in_specs=[pl.BlockSpec(memory_space=pl.ANY)],
        out_specs=pl.BlockSpec(memory_space=pl.ANY),
        scratch_shapes=[
            pltpu.VMEM((2, C, W), jnp.float32),
            pltpu.VMEM((2, C, W), jnp.float32),
            pltpu.VMEM((C, W), jnp.float32),
            pltpu.SemaphoreType.DMA((2,)),
            pltpu.SemaphoreType.DMA((2,)),
            pltpu.SemaphoreType.DMA((4, 3)),
            pltpu.SemaphoreType.DMA((4, 3)),
            pltpu.SemaphoreType.REGULAR,
            pltpu.SemaphoreType.DMA,
            pltpu.SemaphoreType.DMA,
        ],
        compiler_params=pltpu.CompilerParams(collective_id=0),
    )(partial)


def kernel(ids, E):
    my_x = lax.axis_index("x")
    my_y = lax.axis_index("y")
    my_z = lax.axis_index("z")
    local = ids - my_z * V_SHARD
    my_q = my_x * 2 + lax.rem(my_y, 2)
    e_half = lax.dynamic_slice_in_dim(E, my_q * W, W, axis=1)
    one_hot = (local[:, None] == lax.iota(jnp.int32, V_SHARD)[None, :])
    partial = jnp.dot(
        one_hot.astype(jnp.bfloat16), e_half.astype(jnp.bfloat16),
        preferred_element_type=jnp.float32)
    return _allreduce(partial)


# device time: 301508 ns/iter; 11.2209x vs baseline; 1.0900x over previous
import jax
import jax.numpy as jnp
from jax import lax
from jax.experimental import pallas as pl
from jax.experimental.pallas import tpu as pltpu

N_Z = 4
T = 4096
D = 2048
V_SHARD = 8192
C = T // N_Z
W = D // 4


def _embed_allreduce(local_ids, e_half):

    def body(ids_ref, e_ref, out_hbm, send_buf, recv_buf, x_buf, oh_buf,
             send_sems, recv_sems, x_send_sems, x_recv_sems,
             credit_sem, store_sem):
        my_x = lax.axis_index("x")
        my_y = lax.axis_index("y")
        my_z = lax.axis_index("z")
        right = lax.rem(my_z + 1, N_Z)
        left = lax.rem(my_z + N_Z - 1, N_Z)
        ybit = lax.rem(my_y, 2)
        yp = my_y + 1 - 2 * ybit
        my_q = my_x * 2 + ybit
        partners = (
            ((1 - my_x, my_y, my_z), (1 - my_x) * 2 + ybit),
            ((my_x, yp, my_z), my_x * 2 + (1 - ybit)),
            ((1 - my_x, yp, my_z), (1 - my_x) * 2 + (1 - ybit)),
        )

        barrier = pltpu.get_barrier_semaphore()
        for dev in ((my_x, my_y, left), (my_x, my_y, right),
                    *(p[0] for p in partners)):
            pl.semaphore_signal(
                barrier, inc=1, device_id=dev,
                device_id_type=pl.DeviceIdType.MESH,
            )
        pl.semaphore_wait(barrier, 5)

        def rows(c):
            return pl.ds(c * C, C)

        my_cols = pl.ds(my_q * W, W)
        iota = lax.broadcasted_iota(jnp.int32, (C, V_SHARD), 1)

        def matmul_chunk(c, slot):
            oh_buf[:, :] = (ids_ref[rows(c), :] == iota).astype(jnp.bfloat16)
            send_buf[slot, :, :] = jnp.dot(
                oh_buf[:, :], e_ref[:, :],
                preferred_element_type=jnp.float32)

        def credit_left():
            pl.semaphore_signal(
                credit_sem, inc=1, device_id=(my_x, my_y, left),
                device_id_type=pl.DeviceIdType.MESH,
            )

        def store_out(src_ref, c):
            cp = pltpu.make_async_copy(
                src_ref, out_hbm.at[rows(c), my_cols], store_sem)
            cp.start()
            cp.wait()

        def send_step(slot, src_ref):
            rdma = pltpu.make_async_remote_copy(
                src_ref=src_ref,
                dst_ref=recv_buf.at[slot],
                send_sem=send_sems.at[slot],
                recv_sem=recv_sems.at[slot],
                device_id=(my_x, my_y, right),
                device_id_type=pl.DeviceIdType.MESH,
            )
            rdma.start()
            return rdma

        x_rdmas = []

        def x_send(j, src_ref, c):
            for t, (dev, _q) in enumerate(partners):
                rdma = pltpu.make_async_remote_copy(
                    src_ref=src_ref,
                    dst_ref=out_hbm.at[rows(c), my_cols],
                    send_sem=x_send_sems.at[j, t],
                    recv_sem=x_recv_sems.at[j, t],
                    device_id=dev,
                    device_id_type=pl.DeviceIdType.MESH,
                )
                rdma.start()
                x_rdmas.append(rdma)

        rdmas = {}
        for s in range(N_Z):
            slot = s % 2
            chunk = lax.rem(my_z + 2 * N_Z - s, N_Z)
            if s >= 2:
                rdmas[s - 2].wait_send()
            matmul_chunk(chunk, slot)
            if s > 0:
                rdmas[s - 1].wait_recv()
                send_buf[slot, :, :] = (
                    send_buf[slot, :, :] + recv_buf[(s - 1) % 2, :, :])
                credit_left()
            if s >= 2:
                pl.semaphore_wait(credit_sem, 1)
            rdmas[s] = send_step(slot, send_buf.at[slot])
            if s == N_Z - 1:
                x_send(0, send_buf.at[slot], chunk)
                store_out(send_buf.at[slot], chunk)

        for s in range(N_Z, 2 * N_Z - 2):
            slot = s % 2
            prev = (s - 1) % 2
            c_fwd = lax.rem(my_z + 2 * N_Z - (s - 4), N_Z)
            rdmas[s - 1].wait_recv()
            pl.semaphore_wait(credit_sem, 1)
            rdmas[s] = send_step(slot, recv_buf.at[prev])
            if s == N_Z:
                x_buf[:, :] = recv_buf[prev, :, :]
                x_send(1, x_buf, c_fwd)
            else:
                x_send(2, recv_buf.at[prev], c_fwd)
            store_out(recv_buf.at[prev], c_fwd)
            if s < 2 * N_Z - 3:
                rdmas[s].wait_send()
                credit_left()

        last = 2 * N_Z - 3
        rdmas[last].wait_recv()
        final_slot = last % 2
        c_last = lax.rem(my_z + 2, N_Z)
        x_send(3, recv_buf.at[final_slot], c_last)
        store_out(recv_buf.at[final_slot], c_last)

        for s in (2, 3, last):
            rdmas[s].wait_send()
        for rdma in x_rdmas:
            rdma.wait_send()
        c_sched = [lax.rem(my_z + k, N_Z) for k in (1, 0, 3, 2)]
        for t, (dev, q) in enumerate(partners):
            for j, c in enumerate(c_sched):
                recv = pltpu.make_async_remote_copy(
                    src_ref=x_buf,
                    dst_ref=out_hbm.at[rows(c), pl.ds(q * W, W)],
                    send_sem=x_send_sems.at[j, t],
                    recv_sem=x_recv_sems.at[j, t],
                    device_id=dev,
                    device_id_type=pl.DeviceIdType.MESH,
                )
                recv.wait_recv()

    return pl.pallas_call(
        body,
        out_shape=jax.ShapeDtypeStruct((T, D), jnp.float32),
        in_specs=[
            pl.BlockSpec(memory_space=pltpu.MemorySpace.VMEM),
            pl.BlockSpec(memory_space=pltpu.MemorySpace.VMEM),
        ],
        out_specs=pl.BlockSpec(memory_space=pl.ANY),
        scratch_shapes=[
            pltpu.VMEM((2, C, W), jnp.float32),
            pltpu.VMEM((2, C, W), jnp.float32),
            pltpu.VMEM((C, W), jnp.float32),
            pltpu.VMEM((C, V_SHARD), jnp.bfloat16),
            pltpu.SemaphoreType.DMA((2,)),
            pltpu.SemaphoreType.DMA((2,)),
            pltpu.SemaphoreType.DMA((4, 3)),
            pltpu.SemaphoreType.DMA((4, 3)),
            pltpu.SemaphoreType.REGULAR,
            pltpu.SemaphoreType.DMA,
        ],
        compiler_params=pltpu.CompilerParams(collective_id=0),
    )(local_ids, e_half)


def kernel(ids, E):
    my_x = lax.axis_index("x")
    my_y = lax.axis_index("y")
    my_z = lax.axis_index("z")
    local = (ids - my_z * V_SHARD).reshape(T, 1)
    my_q = my_x * 2 + lax.rem(my_y, 2)
    e_half = lax.dynamic_slice_in_dim(E, my_q * W, W, axis=1)
    return _embed_allreduce(local, e_half.astype(jnp.bfloat16))


# device time: 193837 ns/iter; 17.4539x vs baseline; 1.5555x over previous
import jax
import jax.numpy as jnp
from jax import lax
from jax.experimental import pallas as pl
from jax.experimental.pallas import tpu as pltpu

N_Z = 4
T = 4096
D = 2048
V_SHARD = 8192
C = T // N_Z
W = D // 4


def _embed_allreduce(local_ids, e_half):

    def body(ids_ref, e_ref, out_hbm, send_buf, recv_buf, x_buf, oh_buf,
             xy_recv_buf, st_buf,
             send_sems, recv_sems, x_send_sems, x_recv_sems,
             credit_sem, store_sem):
        my_x = lax.axis_index("x")
        my_y = lax.axis_index("y")
        my_z = lax.axis_index("z")
        right = lax.rem(my_z + 1, N_Z)
        left = lax.rem(my_z + N_Z - 1, N_Z)
        ybit = lax.rem(my_y, 2)
        yp = my_y + 1 - 2 * ybit
        my_q = my_x * 2 + ybit
        partners = (
            ((1 - my_x, my_y, my_z), (1 - my_x) * 2 + ybit),
            ((my_x, yp, my_z), my_x * 2 + (1 - ybit)),
            ((1 - my_x, yp, my_z), (1 - my_x) * 2 + (1 - ybit)),
        )

        barrier = pltpu.get_barrier_semaphore()
        for dev in ((my_x, my_y, left), (my_x, my_y, right),
                    *(p[0] for p in partners)):
            pl.semaphore_signal(
                barrier, inc=1, device_id=dev,
                device_id_type=pl.DeviceIdType.MESH,
            )
        pl.semaphore_wait(barrier, 5)

        def rows(c):
            return pl.ds(c * C, C)

        my_cols = pl.ds(my_q * W, W)
        iota = lax.broadcasted_iota(jnp.int32, (C, V_SHARD), 1)

        def matmul_chunk(c, slot):
            oh_buf[:, :] = (ids_ref[rows(c), :] == iota).astype(jnp.bfloat16)
            send_buf[slot, :, :] = jnp.dot(
                oh_buf[:, :], e_ref[:, :],
                preferred_element_type=jnp.float32).astype(jnp.bfloat16)

        def credit_left():
            pl.semaphore_signal(
                credit_sem, inc=1, device_id=(my_x, my_y, left),
                device_id_type=pl.DeviceIdType.MESH,
            )

        def store_out(src_ref, c, cols=None):
            st_buf[:, :] = src_ref[:, :].astype(jnp.float32)
            cp = pltpu.make_async_copy(
                st_buf,
                out_hbm.at[rows(c), my_cols if cols is None else cols],
                store_sem)
            cp.start()
            cp.wait()

        def send_step(slot, src_ref):
            rdma = pltpu.make_async_remote_copy(
                src_ref=src_ref,
                dst_ref=recv_buf.at[slot],
                send_sem=send_sems.at[slot],
                recv_sem=recv_sems.at[slot],
                device_id=(my_x, my_y, right),
                device_id_type=pl.DeviceIdType.MESH,
            )
            rdma.start()
            return rdma

        x_rdmas = []

        def x_send(j, src_ref, c):
            del c
            for t, (dev, _q) in enumerate(partners):
                rdma = pltpu.make_async_remote_copy(
                    src_ref=src_ref,
                    dst_ref=xy_recv_buf.at[j, t],
                    send_sem=x_send_sems.at[j, t],
                    recv_sem=x_recv_sems.at[j, t],
                    device_id=dev,
                    device_id_type=pl.DeviceIdType.MESH,
                )
                rdma.start()
                x_rdmas.append(rdma)

        rdmas = {}
        for s in range(N_Z):
            slot = s % 2
            chunk = lax.rem(my_z + 2 * N_Z - s, N_Z)
            if s >= 2:
                rdmas[s - 2].wait_send()
            matmul_chunk(chunk, slot)
            if s > 0:
                rdmas[s - 1].wait_recv()
                send_buf[slot, :, :] = (
                    send_buf[slot, :, :] + recv_buf[(s - 1) % 2, :, :])
                credit_left()
            if s >= 2:
                pl.semaphore_wait(credit_sem, 1)
            rdmas[s] = send_step(slot, send_buf.at[slot])
            if s == N_Z - 1:
                x_send(0, send_buf.at[slot], chunk)
                store_out(send_buf.at[slot], chunk)

        for s in range(N_Z, 2 * N_Z - 2):
            slot = s % 2
            prev = (s - 1) % 2
            c_fwd = lax.rem(my_z + 2 * N_Z - (s - 4), N_Z)
            rdmas[s - 1].wait_recv()
            pl.semaphore_wait(credit_sem, 1)
            rdmas[s] = send_step(slot, recv_buf.at[prev])
            if s == N_Z:
                x_buf[:, :] = recv_buf[prev, :, :]
                x_send(1, x_buf, c_fwd)
            else:
                x_send(2, recv_buf.at[prev], c_fwd)
            store_out(recv_buf.at[prev], c_fwd)
            if s < 2 * N_Z - 3:
                rdmas[s].wait_send()
                credit_left()

        last = 2 * N_Z - 3
        rdmas[last].wait_recv()
        final_slot = last % 2
        c_last = lax.rem(my_z + 2, N_Z)
        x_send(3, recv_buf.at[final_slot], c_last)
        store_out(recv_buf.at[final_slot], c_last)

        for s in (2, 3, last):
            rdmas[s].wait_send()
        for rdma in x_rdmas:
            rdma.wait_send()
        c_sched = [lax.rem(my_z + k, N_Z) for k in (1, 0, 3, 2)]
        for j, c in enumerate(c_sched):
            for t, (dev, q) in enumerate(partners):
                recv = pltpu.make_async_remote_copy(
                    src_ref=x_buf,
                    dst_ref=xy_recv_buf.at[j, t],
                    send_sem=x_send_sems.at[j, t],
                    recv_sem=x_recv_sems.at[j, t],
                    device_id=dev,
                    device_id_type=pl.DeviceIdType.MESH,
                )
                recv.wait_recv()
                store_out(xy_recv_buf.at[j, t], c, cols=pl.ds(q * W, W))

    return pl.pallas_call(
        body,
        out_shape=jax.ShapeDtypeStruct((T, D), jnp.float32),
        in_specs=[
            pl.BlockSpec(memory_space=pltpu.MemorySpace.VMEM),
            pl.BlockSpec(memory_space=pltpu.MemorySpace.VMEM),
        ],
        out_specs=pl.BlockSpec(memory_space=pl.ANY),
        scratch_shapes=[
            pltpu.VMEM((2, C, W), jnp.bfloat16),
            pltpu.VMEM((2, C, W), jnp.bfloat16),
            pltpu.VMEM((C, W), jnp.bfloat16),
            pltpu.VMEM((C, V_SHARD), jnp.bfloat16),
            pltpu.VMEM((4, 3, C, W), jnp.bfloat16),
            pltpu.VMEM((C, W), jnp.float32),
            pltpu.SemaphoreType.DMA((2,)),
            pltpu.SemaphoreType.DMA((2,)),
            pltpu.SemaphoreType.DMA((4, 3)),
            pltpu.SemaphoreType.DMA((4, 3)),
            pltpu.SemaphoreType.REGULAR,
            pltpu.SemaphoreType.DMA,
        ],
        compiler_params=pltpu.CompilerParams(collective_id=0),
    )(local_ids, e_half)


def kernel(ids, E):
    my_x = lax.axis_index("x")
    my_y = lax.axis_index("y")
    my_z = lax.axis_index("z")
    local = (ids - my_z * V_SHARD).reshape(T, 1)
    my_q = my_x * 2 + lax.rem(my_y, 2)
    e_half = lax.dynamic_slice_in_dim(E, my_q * W, W, axis=1)
    return _embed_allreduce(local, e_half.astype(jnp.bfloat16))
